# baseline (speedup 1.0000x reference)
import functools

import jax
import jax.numpy as jnp
from jax import lax
from jax.experimental import pallas as pl
from jax.experimental.pallas import tpu as pltpu
from jax.experimental.pallas import tpu_sc as plsc

HC = 128
NG = 50
NI = 6
CUTOFF = 10.0

_PREC = lax.Precision.HIGHEST
_PREC_EDGE = lax.Precision.DEFAULT


def _ea_block(eap_ref, ea_ref):
    a = eap_ref[...]
    length = jnp.sqrt(jnp.sum(a * a, axis=1, keepdims=True))
    step = CUTOFF / (NG - 1)
    coeff = -0.5 / step**2
    off = lax.broadcasted_iota(jnp.int32, (1, NG), 1).astype(jnp.float32) * step
    d = length - off
    ea_ref[...] = jnp.exp(coeff * d * d)


def _compute_ea(edge_attr):
    E = edge_attr.shape[0]
    eap = jnp.pad(edge_attr, ((0, 0), (0, 5)))
    Eb = 2000
    return pl.pallas_call(
        _ea_block,
        grid=(E // Eb,),
        in_specs=[pl.BlockSpec((Eb, 8), lambda i: (i, 0))],
        out_specs=pl.BlockSpec((Eb, NG), lambda i: (i, 0)),
        out_shape=jax.ShapeDtypeStruct((E, NG), jnp.float32),
    )(eap)


def _emb_block(x_ref, emb_ref, h_ref):
    xi = x_ref[...]
    nd = emb_ref.shape[0]
    ids = lax.broadcasted_iota(jnp.int32, (xi.shape[0], nd), 1)
    onehot = (ids == xi).astype(jnp.float32)
    h_ref[...] = jnp.dot(onehot, emb_ref[...], precision=_PREC,
                         preferred_element_type=jnp.float32)


def _embed(x2, emb):
    N = x2.shape[0]
    Nb = 2000
    nd = emb.shape[0]
    return pl.pallas_call(
        _emb_block,
        grid=(N // Nb,),
        in_specs=[pl.BlockSpec((Nb, 1), lambda i: (i, 0)),
                  pl.BlockSpec((nd, HC), lambda i: (0, 0))],
        out_specs=pl.BlockSpec((Nb, HC), lambda i: (i, 0)),
        out_shape=jax.ShapeDtypeStruct((N, HC), jnp.float32),
    )(x2, emb)


def _filter_block(ea_ref, w1_ref, b1_ref, w2_ref, b2_ref, w_ref):
    t = jnp.dot(ea_ref[...], w1_ref[...], precision=_PREC_EDGE,
                preferred_element_type=jnp.float32)
    t = jnp.maximum(t + b1_ref[...], 0.0)
    w_ref[...] = jnp.dot(t, w2_ref[...], precision=_PREC_EDGE,
                         preferred_element_type=jnp.float32) + b2_ref[...]


def _filter(ea, w1, b1, w2, b2):
    E = ea.shape[0]
    Eb = 2000
    return pl.pallas_call(
        _filter_block,
        grid=(E // Eb,),
        in_specs=[pl.BlockSpec((Eb, NG), lambda i: (i, 0)),
                  pl.BlockSpec((NG, HC), lambda i: (0, 0)),
                  pl.BlockSpec((1, HC), lambda i: (0, 0)),
                  pl.BlockSpec((HC, HC), lambda i: (0, 0)),
                  pl.BlockSpec((1, HC), lambda i: (0, 0))],
        out_specs=pl.BlockSpec((Eb, HC), lambda i: (i, 0)),
        out_shape=jax.ShapeDtypeStruct((E, HC), jnp.float32),
    )(ea, w1, b1, w2, b2)


def _upd_block(h_ref, p_ref, w1_ref, b1_ref, w2_ref, b2_ref, o_ref):
    aggr = p_ref[0] + p_ref[1]
    t = jnp.maximum(jnp.dot(aggr, w1_ref[...], precision=_PREC,
                            preferred_element_type=jnp.float32) + b1_ref[...],
                    0.0)
    out = jnp.dot(t, w2_ref[...], precision=_PREC,
                  preferred_element_type=jnp.float32) + b2_ref[...]
    o_ref[...] = h_ref[...] + out


def _update(h, parts, w1, b1, w2, b2):
    N = h.shape[0]
    Nb = 2000
    return pl.pallas_call(
        _upd_block,
        grid=(N // Nb,),
        in_specs=[pl.BlockSpec((Nb, HC), lambda i: (i, 0)),
                  pl.BlockSpec((2, Nb, HC), lambda i: (0, i, 0)),
                  pl.BlockSpec((HC, HC), lambda i: (0, 0)),
                  pl.BlockSpec((1, HC), lambda i: (0, 0)),
                  pl.BlockSpec((HC, HC), lambda i: (0, 0)),
                  pl.BlockSpec((1, HC), lambda i: (0, 0))],
        out_specs=pl.BlockSpec((Nb, HC), lambda i: (i, 0)),
        out_shape=jax.ShapeDtypeStruct((N, HC), jnp.float32),
    )(h, parts, w1, b1, w2, b2)


def _ro_block(h_ref, w1_ref, b1_ref, w2_ref, b2_ref, w3_ref, b3_ref, s_ref):
    r = jnp.maximum(jnp.dot(h_ref[...], w1_ref[...], precision=_PREC,
                            preferred_element_type=jnp.float32) + b1_ref[...],
                    0.0)
    r = jnp.maximum(jnp.dot(r, w2_ref[...], precision=_PREC,
                            preferred_element_type=jnp.float32) + b2_ref[...],
                    0.0)
    s_ref[...] = jnp.dot(r, w3_ref[...], precision=_PREC,
                         preferred_element_type=jnp.float32) + b3_ref[...]


def _readout(h, w1, b1, w2, b2, w3, b3):
    N = h.shape[0]
    Nb = 2000
    rh = w1.shape[1]
    rh2 = w2.shape[1]
    return pl.pallas_call(
        _ro_block,
        grid=(N // Nb,),
        in_specs=[pl.BlockSpec((Nb, HC), lambda i: (i, 0)),
                  pl.BlockSpec((HC, rh), lambda i: (0, 0)),
                  pl.BlockSpec((1, rh), lambda i: (0, 0)),
                  pl.BlockSpec((rh, rh2), lambda i: (0, 0)),
                  pl.BlockSpec((1, rh2), lambda i: (0, 0)),
                  pl.BlockSpec((rh2, 1), lambda i: (0, 0)),
                  pl.BlockSpec((1, 1), lambda i: (0, 0))],
        out_specs=pl.BlockSpec((Nb, 1), lambda i: (i, 0)),
        out_shape=jax.ShapeDtypeStruct((N, 1), jnp.float32),
    )(h, w1, b1, w2, b2, w3, b3)


_NW = 32
_FC = 80


def _sc_msg_aggr(h, w_edge, row_f, col3, zero, Np):
    E = w_edge.shape[0]
    pt = E // _NW
    nch = pt // _FC
    npair = nch // 2
    assert nch % 2 == 1
    rpt = Np // 16
    mesh = plsc.VectorSubcoreMesh(core_axis_name="c", subcore_axis_name="s")

    @functools.partial(
        pl.kernel,
        out_type=jax.ShapeDtypeStruct((2 * Np, HC), jnp.float32),
        mesh=mesh,
        scratch_types=[
            pltpu.VMEM((pt,), jnp.int32),
            pltpu.VMEM((_FC, HC), jnp.float32),
            pltpu.VMEM((_FC, HC), jnp.float32),
            pltpu.VMEM((1, _FC), jnp.int32),
            pltpu.VMEM((1, _FC), jnp.int32),
            pltpu.VMEM((_FC, HC), jnp.float32),
            pltpu.VMEM_SHARED((Np, HC), jnp.float32),
            pltpu.SemaphoreType.DMA,
            pltpu.SemaphoreType.DMA,
            pltpu.SemaphoreType.DMA,
        ],
    )
    def k(h_hbm, w_hbm, row_hbm, col_hbm, zero_hbm, out_hbm,
          ridx_v, hg0_v, hg1_v, ci0_v, ci1_v, w_v, acc_sh,
          sem_g, sem_c, sem_w):
        cid = lax.axis_index("c")
        sid = lax.axis_index("s")
        wid = sid * 2 + cid
        r0 = sid * rpt
        base = wid * pt
        pltpu.sync_copy(zero_hbm.at[pl.ds(r0, rpt)], acc_sh.at[pl.ds(r0, rpt)])
        pltpu.sync_copy(row_hbm.at[pl.ds(base, pt)], ridx_v)
        plsc.subcore_barrier()

        hgs = (hg0_v, hg1_v)
        cis = (ci0_v, ci1_v)

        def fire(c, p):
            pltpu.async_copy(h_hbm.at[ridx_v.at[pl.ds(c * _FC, _FC)]],
                             hgs[p], sem_g)
            pltpu.async_copy(col_hbm.at[wid, pl.ds(c, 1)], cis[p], sem_c)

        def fire_w(c):
            pltpu.async_copy(w_hbm.at[pl.ds(base + c * _FC, _FC)], w_v, sem_w)

        def wait_gcw(p):
            pltpu.make_async_copy(h_hbm.at[pl.ds(0, _FC)], hgs[p], sem_g).wait()
            pltpu.make_async_copy(col_hbm.at[0, pl.ds(0, 1)], cis[p],
                                  sem_c).wait()
            pltpu.make_async_copy(w_hbm.at[pl.ds(0, _FC)], w_v, sem_w).wait()

        def mult(p):
            hg = hgs[p]

            @pl.loop(0, _FC, step=8)
            def _(r):
                for ro in range(8):
                    for g in range(HC // 16):
                        slc = (pl.ds(r + ro, 1), pl.ds(g * 16, 16))
                        hg.at[slc][...] = hg.at[slc][...] * w_v.at[slc][...]

        def scat(p):
            pltpu.sync_copy(hgs[p], acc_sh.at[cis[p].at[0]], add=True)

        fire(0, 0)
        fire_w(0)

        @pl.loop(0, npair)
        def _(kp):
            c0 = 2 * kp
            fire(c0 + 1, 1)
            wait_gcw(0)
            mult(0)
            fire_w(c0 + 1)
            scat(0)
            fire(c0 + 2, 0)
            wait_gcw(1)
            mult(1)
            fire_w(c0 + 2)
            scat(1)

        wait_gcw(0)
        mult(0)
        scat(0)

        plsc.subcore_barrier()
        pltpu.sync_copy(acc_sh.at[pl.ds(r0, rpt)],
                        out_hbm.at[pl.ds(cid * Np + r0, rpt)])

    return k(h, w_edge, row_f, col3, zero)


def kernel(x, edge_index, edge_attr, u, emb, fw1, fb1, fw2, fb2,
           d1w, d1b, d2w, d2b, rw1, rb1, rw2, rb2, rw3, rb3):
    N = x.shape[0]
    E = edge_index.shape[1]
    x2 = x.astype(jnp.int32).reshape(N, 1)
    pt = E // _NW
    nch = pt // _FC
    row_f = edge_index[0].astype(jnp.int32)
    col3 = edge_index[1].astype(jnp.int32).reshape(_NW, nch, _FC)

    ea = _compute_ea(edge_attr)
    h = _embed(x2, emb)
    Np = ((N + 127) // 128) * 128
    zero = jnp.zeros((Np, HC), jnp.float32)

    for i in range(NI):
        w_edge = _filter(ea, fw1[i], fb1[i].reshape(1, HC),
                         fw2[i], fb2[i].reshape(1, HC))
        parts = _sc_msg_aggr(h, w_edge, row_f, col3, zero, Np)
        h = _update(h, parts.reshape(2, Np, HC), d1w[i], d1b[i].reshape(1, HC),
                    d2w[i], d2b[i].reshape(1, HC))

    shifts = _readout(h, rw1, rb1.reshape(1, -1), rw2, rb2.reshape(1, -1),
                      rw3, rb3.reshape(1, 1))
    return (shifts, (h, ea, u))


# speedup vs baseline: 1.0150x; 1.0150x over previous
import functools

import jax
import jax.numpy as jnp
from jax import lax
from jax.experimental import pallas as pl
from jax.experimental.pallas import tpu as pltpu
from jax.experimental.pallas import tpu_sc as plsc

HC = 128
NG = 50
NI = 6
CUTOFF = 10.0

_PREC = lax.Precision.HIGHEST
_PREC_EDGE = lax.Precision.DEFAULT


def _ea_block(eap_ref, ea_ref):
    a = eap_ref[...]
    length = jnp.sqrt(jnp.sum(a * a, axis=1, keepdims=True))
    step = CUTOFF / (NG - 1)
    coeff = -0.5 / step**2
    off = lax.broadcasted_iota(jnp.int32, (1, NG), 1).astype(jnp.float32) * step
    d = length - off
    ea_ref[...] = jnp.exp(coeff * d * d)


def _compute_ea(edge_attr):
    E = edge_attr.shape[0]
    eap = jnp.pad(edge_attr, ((0, 0), (0, 5)))
    Eb = 2000
    return pl.pallas_call(
        _ea_block,
        grid=(E // Eb,),
        in_specs=[pl.BlockSpec((Eb, 8), lambda i: (i, 0))],
        out_specs=pl.BlockSpec((Eb, NG), lambda i: (i, 0)),
        out_shape=jax.ShapeDtypeStruct((E, NG), jnp.float32),
    )(eap)


def _emb_block(x_ref, emb_ref, h_ref):
    xi = x_ref[...]
    nd = emb_ref.shape[0]
    ids = lax.broadcasted_iota(jnp.int32, (xi.shape[0], nd), 1)
    onehot = (ids == xi).astype(jnp.float32)
    h_ref[...] = jnp.dot(onehot, emb_ref[...], precision=_PREC,
                         preferred_element_type=jnp.float32)


def _embed(x2, emb):
    N = x2.shape[0]
    Nb = 2000
    nd = emb.shape[0]
    return pl.pallas_call(
        _emb_block,
        grid=(N // Nb,),
        in_specs=[pl.BlockSpec((Nb, 1), lambda i: (i, 0)),
                  pl.BlockSpec((nd, HC), lambda i: (0, 0))],
        out_specs=pl.BlockSpec((Nb, HC), lambda i: (i, 0)),
        out_shape=jax.ShapeDtypeStruct((N, HC), jnp.float32),
    )(x2, emb)


def _filter_block(ea_ref, w1_ref, b1_ref, w2_ref, b2_ref, w_ref):
    t = jnp.dot(ea_ref[...], w1_ref[...], precision=_PREC_EDGE,
                preferred_element_type=jnp.float32)
    t = jnp.maximum(t + b1_ref[...], 0.0)
    w_ref[...] = jnp.dot(t, w2_ref[...], precision=_PREC_EDGE,
                         preferred_element_type=jnp.float32) + b2_ref[...]


def _filter(ea, w1, b1, w2, b2):
    E = ea.shape[0]
    Eb = 2000
    return pl.pallas_call(
        _filter_block,
        grid=(E // Eb,),
        in_specs=[pl.BlockSpec((Eb, NG), lambda i: (i, 0)),
                  pl.BlockSpec((NG, HC), lambda i: (0, 0)),
                  pl.BlockSpec((1, HC), lambda i: (0, 0)),
                  pl.BlockSpec((HC, HC), lambda i: (0, 0)),
                  pl.BlockSpec((1, HC), lambda i: (0, 0))],
        out_specs=pl.BlockSpec((Eb, HC), lambda i: (i, 0)),
        out_shape=jax.ShapeDtypeStruct((E, HC), jnp.float32),
    )(ea, w1, b1, w2, b2)


def _upd_block(h_ref, p_ref, w1_ref, b1_ref, w2_ref, b2_ref, o_ref):
    aggr = p_ref[0] + p_ref[1]
    t = jnp.maximum(jnp.dot(aggr, w1_ref[...], precision=_PREC,
                            preferred_element_type=jnp.float32) + b1_ref[...],
                    0.0)
    out = jnp.dot(t, w2_ref[...], precision=_PREC,
                  preferred_element_type=jnp.float32) + b2_ref[...]
    o_ref[...] = h_ref[...] + out


def _update(h, parts, w1, b1, w2, b2):
    N = h.shape[0]
    Nb = 2000
    return pl.pallas_call(
        _upd_block,
        grid=(N // Nb,),
        in_specs=[pl.BlockSpec((Nb, HC), lambda i: (i, 0)),
                  pl.BlockSpec((2, Nb, HC), lambda i: (0, i, 0)),
                  pl.BlockSpec((HC, HC), lambda i: (0, 0)),
                  pl.BlockSpec((1, HC), lambda i: (0, 0)),
                  pl.BlockSpec((HC, HC), lambda i: (0, 0)),
                  pl.BlockSpec((1, HC), lambda i: (0, 0))],
        out_specs=pl.BlockSpec((Nb, HC), lambda i: (i, 0)),
        out_shape=jax.ShapeDtypeStruct((N, HC), jnp.float32),
    )(h, parts, w1, b1, w2, b2)


def _ro_block(h_ref, w1_ref, b1_ref, w2_ref, b2_ref, w3_ref, b3_ref, s_ref):
    r = jnp.maximum(jnp.dot(h_ref[...], w1_ref[...], precision=_PREC,
                            preferred_element_type=jnp.float32) + b1_ref[...],
                    0.0)
    r = jnp.maximum(jnp.dot(r, w2_ref[...], precision=_PREC,
                            preferred_element_type=jnp.float32) + b2_ref[...],
                    0.0)
    s_ref[...] = jnp.dot(r, w3_ref[...], precision=_PREC,
                         preferred_element_type=jnp.float32) + b3_ref[...]


def _readout(h, w1, b1, w2, b2, w3, b3):
    N = h.shape[0]
    Nb = 2000
    rh = w1.shape[1]
    rh2 = w2.shape[1]
    return pl.pallas_call(
        _ro_block,
        grid=(N // Nb,),
        in_specs=[pl.BlockSpec((Nb, HC), lambda i: (i, 0)),
                  pl.BlockSpec((HC, rh), lambda i: (0, 0)),
                  pl.BlockSpec((1, rh), lambda i: (0, 0)),
                  pl.BlockSpec((rh, rh2), lambda i: (0, 0)),
                  pl.BlockSpec((1, rh2), lambda i: (0, 0)),
                  pl.BlockSpec((rh2, 1), lambda i: (0, 0)),
                  pl.BlockSpec((1, 1), lambda i: (0, 0))],
        out_specs=pl.BlockSpec((Nb, 1), lambda i: (i, 0)),
        out_shape=jax.ShapeDtypeStruct((N, 1), jnp.float32),
    )(h, w1, b1, w2, b2, w3, b3)


_NW = 32
_FC = 80


def _sc_msg_aggr(h, w_edge, row_f, col3, zero, Np):
    E = w_edge.shape[0]
    pt = E // _NW
    nch = pt // _FC
    npair = nch // 2
    assert nch % 2 == 1
    rpt = Np // 16
    mesh = plsc.VectorSubcoreMesh(core_axis_name="c", subcore_axis_name="s")

    @functools.partial(
        pl.kernel,
        out_type=jax.ShapeDtypeStruct((2 * Np, HC), jnp.float32),
        mesh=mesh,
        scratch_types=[
            pltpu.VMEM((pt,), jnp.int32),
            pltpu.VMEM((_FC, HC), jnp.float32),
            pltpu.VMEM((_FC, HC), jnp.float32),
            pltpu.VMEM((1, _FC), jnp.int32),
            pltpu.VMEM((1, _FC), jnp.int32),
            pltpu.VMEM((_FC, HC), jnp.float32),
            pltpu.VMEM_SHARED((Np, HC), jnp.float32),
            pltpu.SemaphoreType.DMA,
            pltpu.SemaphoreType.DMA,
            pltpu.SemaphoreType.DMA,
        ],
    )
    def k(h_hbm, w_hbm, row_hbm, col_hbm, zero_hbm, out_hbm,
          ridx_v, hg0_v, hg1_v, ci0_v, ci1_v, w_v, acc_sh,
          sem_g, sem_c, sem_w):
        cid = lax.axis_index("c")
        sid = lax.axis_index("s")
        wid = sid * 2 + cid
        r0 = sid * rpt
        base = wid * pt
        pltpu.sync_copy(zero_hbm.at[pl.ds(r0, rpt)], acc_sh.at[pl.ds(r0, rpt)])
        pltpu.sync_copy(row_hbm.at[pl.ds(base, pt)], ridx_v)
        plsc.subcore_barrier()

        hgs = (hg0_v, hg1_v)
        cis = (ci0_v, ci1_v)

        def fire(c, p):
            pltpu.async_copy(h_hbm.at[ridx_v.at[pl.ds(c * _FC, _FC)]],
                             hgs[p], sem_g)
            pltpu.async_copy(col_hbm.at[wid, pl.ds(c, 1)], cis[p], sem_c)

        def fire_w(c):
            pltpu.async_copy(w_hbm.at[pl.ds(base + c * _FC, _FC)], w_v, sem_w)

        def wait_gcw(p):
            pltpu.make_async_copy(h_hbm.at[pl.ds(0, _FC)], hgs[p], sem_g).wait()
            pltpu.make_async_copy(col_hbm.at[0, pl.ds(0, 1)], cis[p],
                                  sem_c).wait()
            pltpu.make_async_copy(w_hbm.at[pl.ds(0, _FC)], w_v, sem_w).wait()

        def mult(p):
            hg = hgs[p]

            @pl.loop(0, _FC)
            def _(r):
                for g in range(HC // 16):
                    slc = (pl.ds(r, 1), pl.ds(g * 16, 16))
                    hg.at[slc][...] = hg.at[slc][...] * w_v.at[slc][...]

        def scat(p):
            pltpu.sync_copy(hgs[p], acc_sh.at[cis[p].at[0]], add=True)

        fire(0, 0)
        fire_w(0)

        @pl.loop(0, npair)
        def _(kp):
            c0 = 2 * kp
            fire(c0 + 1, 1)
            wait_gcw(0)
            mult(0)
            fire_w(c0 + 1)
            scat(0)
            fire(c0 + 2, 0)
            wait_gcw(1)
            mult(1)
            fire_w(c0 + 2)
            scat(1)

        wait_gcw(0)
        mult(0)
        scat(0)

        plsc.subcore_barrier()
        pltpu.sync_copy(acc_sh.at[pl.ds(r0, rpt)],
                        out_hbm.at[pl.ds(cid * Np + r0, rpt)])

    return k(h, w_edge, row_f, col3, zero)


def kernel(x, edge_index, edge_attr, u, emb, fw1, fb1, fw2, fb2,
           d1w, d1b, d2w, d2b, rw1, rb1, rw2, rb2, rw3, rb3):
    N = x.shape[0]
    E = edge_index.shape[1]
    x2 = x.astype(jnp.int32).reshape(N, 1)
    pt = E // _NW
    nch = pt // _FC
    row_f = edge_index[0].astype(jnp.int32)
    col3 = edge_index[1].astype(jnp.int32).reshape(_NW, nch, _FC)

    ea = _compute_ea(edge_attr)
    h = _embed(x2, emb)
    Np = ((N + 127) // 128) * 128
    zero = jnp.zeros((Np, HC), jnp.float32)

    w_edges = [_filter(ea, fw1[i], fb1[i].reshape(1, HC),
                       fw2[i], fb2[i].reshape(1, HC)) for i in range(NI)]
    for i in range(NI):
        parts = _sc_msg_aggr(h, w_edges[i], row_f, col3, zero, Np)
        h = _update(h, parts.reshape(2, Np, HC), d1w[i], d1b[i].reshape(1, HC),
                    d2w[i], d2b[i].reshape(1, HC))

    shifts = _readout(h, rw1, rb1.reshape(1, -1), rw2, rb2.reshape(1, -1),
                      rw3, rb3.reshape(1, 1))
    return (shifts, (h, ea, u))


# speedup vs baseline: 1.0191x; 1.0041x over previous
import functools

import jax
import jax.numpy as jnp
from jax import lax
from jax.experimental import pallas as pl
from jax.experimental.pallas import tpu as pltpu
from jax.experimental.pallas import tpu_sc as plsc

HC = 128
NG = 50
NI = 6
CUTOFF = 10.0

_PREC = lax.Precision.HIGHEST
_PREC_EDGE = lax.Precision.DEFAULT


def _ea_block(eap_ref, ea_ref):
    a = eap_ref[...]
    length = jnp.sqrt(jnp.sum(a * a, axis=1, keepdims=True))
    step = CUTOFF / (NG - 1)
    coeff = -0.5 / step**2
    off = lax.broadcasted_iota(jnp.int32, (1, NG), 1).astype(jnp.float32) * step
    d = length - off
    ea_ref[...] = jnp.exp(coeff * d * d)


def _compute_ea(edge_attr):
    E = edge_attr.shape[0]
    eap = jnp.pad(edge_attr, ((0, 0), (0, 5)))
    Eb = 2000
    return pl.pallas_call(
        _ea_block,
        grid=(E // Eb,),
        in_specs=[pl.BlockSpec((Eb, 8), lambda i: (i, 0))],
        out_specs=pl.BlockSpec((Eb, NG), lambda i: (i, 0)),
        out_shape=jax.ShapeDtypeStruct((E, NG), jnp.float32),
    )(eap)


def _ea_t_block(eapt_ref, eat_ref):
    a = eapt_ref[...]
    length = jnp.sqrt(jnp.sum(a * a, axis=0, keepdims=True))
    step = CUTOFF / (NG - 1)
    coeff = -0.5 / step**2
    off = lax.broadcasted_iota(jnp.int32, (NG, 1), 0).astype(jnp.float32) * step
    d = length - off
    eat_ref[...] = jnp.exp(coeff * d * d)


def _compute_ea_t(edge_attr):
    E = edge_attr.shape[0]
    eapt = jnp.pad(edge_attr, ((0, 0), (0, 5))).T
    Eb = 2560
    return pl.pallas_call(
        _ea_t_block,
        grid=(E // Eb,),
        in_specs=[pl.BlockSpec((8, Eb), lambda i: (0, i))],
        out_specs=pl.BlockSpec((NG, Eb), lambda i: (0, i)),
        out_shape=jax.ShapeDtypeStruct((NG, E), jnp.float32),
    )(eapt)


def _emb_block(x_ref, emb_ref, h_ref):
    xi = x_ref[...]
    nd = emb_ref.shape[0]
    ids = lax.broadcasted_iota(jnp.int32, (xi.shape[0], nd), 1)
    onehot = (ids == xi).astype(jnp.float32)
    h_ref[...] = jnp.dot(onehot, emb_ref[...], precision=_PREC,
                         preferred_element_type=jnp.float32)


def _embed(x2, emb):
    N = x2.shape[0]
    Nb = 2000
    nd = emb.shape[0]
    return pl.pallas_call(
        _emb_block,
        grid=(N // Nb,),
        in_specs=[pl.BlockSpec((Nb, 1), lambda i: (i, 0)),
                  pl.BlockSpec((nd, HC), lambda i: (0, 0))],
        out_specs=pl.BlockSpec((Nb, HC), lambda i: (i, 0)),
        out_shape=jax.ShapeDtypeStruct((N, HC), jnp.float32),
    )(x2, emb)


def _filter_block(eat_ref, w1_ref, b1_ref, w2_ref, b2_ref, w_ref):
    t = lax.dot_general(eat_ref[...], w1_ref[...],
                        (((0,), (0,)), ((), ())), precision=_PREC_EDGE,
                        preferred_element_type=jnp.float32)
    t = jnp.maximum(t + b1_ref[...], 0.0)
    w_ref[...] = jnp.dot(t, w2_ref[...], precision=_PREC_EDGE,
                         preferred_element_type=jnp.float32) + b2_ref[...]


def _filter(ea_t, w1, b1, w2, b2):
    E = ea_t.shape[1]
    Eb = 2560
    return pl.pallas_call(
        _filter_block,
        grid=(E // Eb,),
        in_specs=[pl.BlockSpec((NG, Eb), lambda i: (0, i)),
                  pl.BlockSpec((NG, HC), lambda i: (0, 0)),
                  pl.BlockSpec((1, HC), lambda i: (0, 0)),
                  pl.BlockSpec((HC, HC), lambda i: (0, 0)),
                  pl.BlockSpec((1, HC), lambda i: (0, 0))],
        out_specs=pl.BlockSpec((Eb, HC), lambda i: (i, 0)),
        out_shape=jax.ShapeDtypeStruct((E, HC), jnp.float32),
    )(ea_t, w1, b1, w2, b2)


def _upd_block(h_ref, p_ref, w1_ref, b1_ref, w2_ref, b2_ref, o_ref):
    aggr = p_ref[0] + p_ref[1]
    t = jnp.maximum(jnp.dot(aggr, w1_ref[...], precision=_PREC,
                            preferred_element_type=jnp.float32) + b1_ref[...],
                    0.0)
    out = jnp.dot(t, w2_ref[...], precision=_PREC,
                  preferred_element_type=jnp.float32) + b2_ref[...]
    o_ref[...] = h_ref[...] + out


def _update(h, parts, w1, b1, w2, b2):
    N = h.shape[0]
    Nb = 2000
    return pl.pallas_call(
        _upd_block,
        grid=(N // Nb,),
        in_specs=[pl.BlockSpec((Nb, HC), lambda i: (i, 0)),
                  pl.BlockSpec((2, Nb, HC), lambda i: (0, i, 0)),
                  pl.BlockSpec((HC, HC), lambda i: (0, 0)),
                  pl.BlockSpec((1, HC), lambda i: (0, 0)),
                  pl.BlockSpec((HC, HC), lambda i: (0, 0)),
                  pl.BlockSpec((1, HC), lambda i: (0, 0))],
        out_specs=pl.BlockSpec((Nb, HC), lambda i: (i, 0)),
        out_shape=jax.ShapeDtypeStruct((N, HC), jnp.float32),
    )(h, parts, w1, b1, w2, b2)


def _ro_block(h_ref, w1_ref, b1_ref, w2_ref, b2_ref, w3_ref, b3_ref, s_ref):
    r = jnp.maximum(jnp.dot(h_ref[...], w1_ref[...], precision=_PREC,
                            preferred_element_type=jnp.float32) + b1_ref[...],
                    0.0)
    r = jnp.maximum(jnp.dot(r, w2_ref[...], precision=_PREC,
                            preferred_element_type=jnp.float32) + b2_ref[...],
                    0.0)
    s_ref[...] = jnp.dot(r, w3_ref[...], precision=_PREC,
                         preferred_element_type=jnp.float32) + b3_ref[...]


def _readout(h, w1, b1, w2, b2, w3, b3):
    N = h.shape[0]
    Nb = 2000
    rh = w1.shape[1]
    rh2 = w2.shape[1]
    return pl.pallas_call(
        _ro_block,
        grid=(N // Nb,),
        in_specs=[pl.BlockSpec((Nb, HC), lambda i: (i, 0)),
                  pl.BlockSpec((HC, rh), lambda i: (0, 0)),
                  pl.BlockSpec((1, rh), lambda i: (0, 0)),
                  pl.BlockSpec((rh, rh2), lambda i: (0, 0)),
                  pl.BlockSpec((1, rh2), lambda i: (0, 0)),
                  pl.BlockSpec((rh2, 1), lambda i: (0, 0)),
                  pl.BlockSpec((1, 1), lambda i: (0, 0))],
        out_specs=pl.BlockSpec((Nb, 1), lambda i: (i, 0)),
        out_shape=jax.ShapeDtypeStruct((N, 1), jnp.float32),
    )(h, w1, b1, w2, b2, w3, b3)


_NW = 32
_FC = 80


def _sc_msg_aggr(h, w_edge, row_f, col3, zero, Np):
    E = w_edge.shape[0]
    pt = E // _NW
    nch = pt // _FC
    npair = nch // 2
    assert nch % 2 == 1
    rpt = Np // 16
    mesh = plsc.VectorSubcoreMesh(core_axis_name="c", subcore_axis_name="s")

    @functools.partial(
        pl.kernel,
        out_type=jax.ShapeDtypeStruct((2 * Np, HC), jnp.float32),
        mesh=mesh,
        scratch_types=[
            pltpu.VMEM((pt,), jnp.int32),
            pltpu.VMEM((_FC, HC), jnp.float32),
            pltpu.VMEM((_FC, HC), jnp.float32),
            pltpu.VMEM((1, _FC), jnp.int32),
            pltpu.VMEM((1, _FC), jnp.int32),
            pltpu.VMEM((_FC, HC), jnp.float32),
            pltpu.VMEM_SHARED((Np, HC), jnp.float32),
            pltpu.SemaphoreType.DMA,
            pltpu.SemaphoreType.DMA,
            pltpu.SemaphoreType.DMA,
        ],
    )
    def k(h_hbm, w_hbm, row_hbm, col_hbm, zero_hbm, out_hbm,
          ridx_v, hg0_v, hg1_v, ci0_v, ci1_v, w_v, acc_sh,
          sem_g, sem_c, sem_w):
        cid = lax.axis_index("c")
        sid = lax.axis_index("s")
        wid = sid * 2 + cid
        r0 = sid * rpt
        base = wid * pt
        pltpu.sync_copy(zero_hbm.at[pl.ds(r0, rpt)], acc_sh.at[pl.ds(r0, rpt)])
        pltpu.sync_copy(row_hbm.at[pl.ds(base, pt)], ridx_v)
        plsc.subcore_barrier()

        hgs = (hg0_v, hg1_v)
        cis = (ci0_v, ci1_v)

        def fire(c, p):
            pltpu.async_copy(h_hbm.at[ridx_v.at[pl.ds(c * _FC, _FC)]],
                             hgs[p], sem_g)
            pltpu.async_copy(col_hbm.at[wid, pl.ds(c, 1)], cis[p], sem_c)

        def fire_w(c):
            pltpu.async_copy(w_hbm.at[pl.ds(base + c * _FC, _FC)], w_v, sem_w)

        def wait_gcw(p):
            pltpu.make_async_copy(h_hbm.at[pl.ds(0, _FC)], hgs[p], sem_g).wait()
            pltpu.make_async_copy(col_hbm.at[0, pl.ds(0, 1)], cis[p],
                                  sem_c).wait()
            pltpu.make_async_copy(w_hbm.at[pl.ds(0, _FC)], w_v, sem_w).wait()

        def mult(p):
            hg = hgs[p]

            @pl.loop(0, _FC)
            def _(r):
                for g in range(HC // 16):
                    slc = (pl.ds(r, 1), pl.ds(g * 16, 16))
                    hg.at[slc][...] = hg.at[slc][...] * w_v.at[slc][...]

        def scat(p):
            pltpu.sync_copy(hgs[p], acc_sh.at[cis[p].at[0]], add=True)

        fire(0, 0)
        fire_w(0)

        @pl.loop(0, npair)
        def _(kp):
            c0 = 2 * kp
            fire(c0 + 1, 1)
            wait_gcw(0)
            mult(0)
            fire_w(c0 + 1)
            scat(0)
            fire(c0 + 2, 0)
            wait_gcw(1)
            mult(1)
            fire_w(c0 + 2)
            scat(1)

        wait_gcw(0)
        mult(0)
        scat(0)

        plsc.subcore_barrier()
        pltpu.sync_copy(acc_sh.at[pl.ds(r0, rpt)],
                        out_hbm.at[pl.ds(cid * Np + r0, rpt)])

    return k(h, w_edge, row_f, col3, zero)


def kernel(x, edge_index, edge_attr, u, emb, fw1, fb1, fw2, fb2,
           d1w, d1b, d2w, d2b, rw1, rb1, rw2, rb2, rw3, rb3):
    N = x.shape[0]
    E = edge_index.shape[1]
    x2 = x.astype(jnp.int32).reshape(N, 1)
    pt = E // _NW
    nch = pt // _FC
    row_f = edge_index[0].astype(jnp.int32)
    col3 = edge_index[1].astype(jnp.int32).reshape(_NW, nch, _FC)

    ea = _compute_ea(edge_attr)
    ea_t = _compute_ea_t(edge_attr)
    h = _embed(x2, emb)
    Np = ((N + 127) // 128) * 128
    zero = jnp.zeros((Np, HC), jnp.float32)

    w_edges = [_filter(ea_t, fw1[i], fb1[i].reshape(1, HC),
                       fw2[i], fb2[i].reshape(1, HC)) for i in range(NI)]
    for i in range(NI):
        parts = _sc_msg_aggr(h, w_edges[i], row_f, col3, zero, Np)
        h = _update(h, parts.reshape(2, Np, HC), d1w[i], d1b[i].reshape(1, HC),
                    d2w[i], d2b[i].reshape(1, HC))

    shifts = _readout(h, rw1, rb1.reshape(1, -1), rw2, rb2.reshape(1, -1),
                      rw3, rb3.reshape(1, 1))
    return (shifts, (h, ea, u))
